# Initial kernel scaffold; baseline (speedup 1.0000x reference)
import jax
import jax.numpy as jnp
from jax.experimental import pallas as pl


def kernel(x, embeddings):
    raise NotImplementedError("write your pallas kernel here")


# speedup vs baseline: 18.4562x; 18.4562x over previous
import functools

import jax
import jax.numpy as jnp
import numpy as np
from jax import lax
from jax.experimental import pallas as pl
from jax.experimental.pallas import tpu as pltpu
from jax.experimental.pallas import tpu_sc as plsc

_NUM_LEVELS = 12
_LOG2_HASH = 19
_HASH_SIZE = 2 ** _LOG2_HASH
_BASE_RES = 16
_MAX_RES = 1024
_GROWTH = np.exp((np.log(_MAX_RES) - np.log(_BASE_RES)) / (_NUM_LEVELS - 1))
_RES = [int(_BASE_RES * _GROWTH ** i) for i in range(_NUM_LEVELS)]
_N = 1048576
_F = 2 * _NUM_LEVELS

_P1 = np.int32(2654435761 - (1 << 32))
_P2 = np.int32(805459861)
_MASK = np.int32(_HASH_SIZE - 1)

_NW = 32
_PTS_PER_W = _N // _NW
_C = 1024
_WORDS_PER_C = _C * _F


def _sc_lookup(x0, x1, x2, table):
    mesh = plsc.VectorSubcoreMesh(core_axis_name="c", subcore_axis_name="s")

    @functools.partial(
        pl.kernel,
        mesh=mesh,
        out_type=jax.ShapeDtypeStruct((_N * _F,), jnp.float32),
        scratch_types=[
            pltpu.VMEM((_C,), jnp.float32),
            pltpu.VMEM((_C,), jnp.float32),
            pltpu.VMEM((_C,), jnp.float32),
            pltpu.VMEM((_WORDS_PER_C,), jnp.int32),
            pltpu.VMEM((_WORDS_PER_C,), jnp.float32),
            pltpu.SemaphoreType.DMA,
        ],
        compiler_params=pltpu.CompilerParams(
            needs_layout_passes=False, use_tc_tiling_on_sc=False),
    )
    def k(x0_hbm, x1_hbm, x2_hbm, table_hbm, out_hbm,
          x0_v, x1_v, x2_v, idx_v, rows_v, sem):
        wid = lax.axis_index("s") * np.int32(2) + lax.axis_index("c")
        tile_base = wid * np.int32(_PTS_PER_W)
        zero = wid * np.int32(0)
        lanes24 = lax.iota(jnp.int32, 16) * np.int32(_F)

        @pl.loop(tile_base, tile_base + np.int32(_PTS_PER_W),
                 step=np.int32(_C))
        def chunk_body(base):
            pltpu.sync_copy(x0_hbm.at[pl.ds(base, _C)], x0_v)
            pltpu.sync_copy(x1_hbm.at[pl.ds(base, _C)], x1_v)
            pltpu.sync_copy(x2_hbm.at[pl.ds(base, _C)], x2_v)

            @pl.loop(zero, np.int32(_C), step=np.int32(16))
            def g_body(s):
                xs0 = x0_v[pl.ds(s, 16)]
                xs1 = x1_v[pl.ds(s, 16)]
                xs2 = x2_v[pl.ds(s, 16)]
                pos0 = lanes24 + s * np.int32(_F)
                for i in range(_NUM_LEVELS):
                    r = jnp.float32(_RES[i])
                    a0 = (xs0 * r).astype(jnp.int32)
                    a1 = (xs1 * r).astype(jnp.int32)
                    a2 = (xs2 * r).astype(jnp.int32)
                    h = (a0 ^ (a1 * _P1) ^ (a2 * _P2)) & _MASK
                    w0 = (h << np.int32(1)) | np.int32(i << (_LOG2_HASH + 1))
                    plsc.store_scatter(
                        idx_v, [pos0 + np.int32(2 * i)], w0)
                    plsc.store_scatter(
                        idx_v, [pos0 + np.int32(2 * i + 1)],
                        w0 | np.int32(1))

            pltpu.async_copy(table_hbm.at[idx_v], rows_v, sem).wait()
            pltpu.sync_copy(
                rows_v, out_hbm.at[pl.ds(base * np.int32(_F), _WORDS_PER_C)])

    return k(x0, x1, x2, table)


def kernel(x, embeddings):
    x = x.astype(jnp.float32)
    table = embeddings.astype(jnp.float32).reshape(_NUM_LEVELS * _HASH_SIZE * 2)
    out = _sc_lookup(x[:, 0], x[:, 1], x[:, 2], table)
    return out.reshape(_N, _F)


# speedup vs baseline: 180.3640x; 9.7725x over previous
import functools

import jax
import jax.numpy as jnp
import numpy as np
from jax import lax
from jax.experimental import pallas as pl
from jax.experimental.pallas import tpu as pltpu
from jax.experimental.pallas import tpu_sc as plsc

_NUM_LEVELS = 12
_LOG2_HASH = 19
_HASH_SIZE = 2 ** _LOG2_HASH
_BASE_RES = 16
_MAX_RES = 1024
_GROWTH = np.exp((np.log(_MAX_RES) - np.log(_BASE_RES)) / (_NUM_LEVELS - 1))
_RES = [int(_BASE_RES * _GROWTH ** i) for i in range(_NUM_LEVELS)]
_N = 1048576
_F = 2 * _NUM_LEVELS
_TW = _NUM_LEVELS * _HASH_SIZE * 2

_P1 = np.int32(2654435761 - (1 << 32))
_P2 = np.int32(805459861)
_MASK = np.int32(_HASH_SIZE - 1)

_NW = 32
_PTS_PER_W = _N // _NW
_C = 1024
_WORDS_PER_C = _C * _F
_TRS = _F // 8
_PLANE = (_N // 128) * 1024


def _sc_lookup(x0, x1, x2, table):
    mesh = plsc.VectorSubcoreMesh(core_axis_name="c", subcore_axis_name="s")

    @functools.partial(
        pl.kernel,
        mesh=mesh,
        out_type=jax.ShapeDtypeStruct((_N * _F,), jnp.float32),
        scratch_types=[
            pltpu.VMEM((_C,), jnp.float32), pltpu.VMEM((_C,), jnp.float32),
            pltpu.VMEM((_C,), jnp.float32), pltpu.VMEM((_C,), jnp.float32),
            pltpu.VMEM((_C,), jnp.float32), pltpu.VMEM((_C,), jnp.float32),
            pltpu.VMEM((_WORDS_PER_C,), jnp.int32),
            pltpu.VMEM((_WORDS_PER_C,), jnp.int32),
            pltpu.VMEM((_WORDS_PER_C,), jnp.float32),
            pltpu.VMEM((_WORDS_PER_C,), jnp.float32),
            pltpu.SemaphoreType.DMA,
            pltpu.SemaphoreType.DMA,
            pltpu.SemaphoreType.DMA,
        ],
        compiler_params=pltpu.CompilerParams(
            needs_layout_passes=False, use_tc_tiling_on_sc=False),
    )
    def k(x0_hbm, x1_hbm, x2_hbm, table_hbm, out_hbm,
          x0a, x1a, x2a, x0b, x1b, x2b,
          idx_a, idx_b, rows_a, rows_b, semg, semx, semw):
        wid = lax.axis_index("s") * np.int32(2) + lax.axis_index("c")
        tile_base = wid * np.int32(_PTS_PER_W)
        zero = wid * np.int32(0)
        tile_end = tile_base + np.int32(_PTS_PER_W)
        last_load = tile_end - np.int32(_C)
        lanes = lax.iota(jnp.int32, 16)

        def load_x(base, xv0, xv1, xv2):
            c0 = pltpu.async_copy(x0_hbm.at[pl.ds(base, _C)], xv0, semx)
            c1 = pltpu.async_copy(x1_hbm.at[pl.ds(base, _C)], xv1, semx)
            c2 = pltpu.async_copy(x2_hbm.at[pl.ds(base, _C)], xv2, semx)
            c0.wait(); c1.wait(); c2.wait()

        def compute_idx(xv0, xv1, xv2, idx_v):
            @pl.loop(zero, np.int32(_C), step=np.int32(16))
            def g_body(s):
                xs0 = xv0[pl.ds(s, 16)]
                xs1 = xv1[pl.ds(s, 16)]
                xs2 = xv2[pl.ds(s, 16)]
                t = ((s >> np.int32(7)) << np.int32(10)) | \
                    (s & np.int32(127))
                pv = lanes + t
                for i in range(_NUM_LEVELS):
                    r = jnp.float32(_RES[i])
                    a0 = (xs0 * r).astype(jnp.int32)
                    a1 = (xs1 * r).astype(jnp.int32)
                    a2 = (xs2 * r).astype(jnp.int32)
                    h = (a0 ^ (a1 * _P1) ^ (a2 * _P2)) & _MASK
                    w0 = (((h << np.int32(1)) & np.int32(-256))
                          | (h & np.int32(127))
                          | np.int32(i << 20))
                    w1 = w0 | np.int32(128)
                    f0 = 2 * i
                    pc0 = np.int32((f0 // 8) * 8192 + (f0 % 8) * 128)
                    plsc.store_scatter(idx_v, [pv + pc0], w0)
                    plsc.store_scatter(idx_v, [pv + (pc0 + np.int32(128))],
                                       w1)

        def write_descs(base, rows_v):
            return [
                pltpu.make_async_copy(
                    rows_v.at[pl.ds(np.int32(tr * 8192), 8192)],
                    out_hbm.at[pl.ds(base * np.int32(8)
                                     + np.int32(tr * _PLANE), 8192)],
                    semw)
                for tr in range(_TRS)]

        def issue_writes(base, rows_v):
            for d in write_descs(base, rows_v):
                d.start()

        def wait_writes(base, rows_v):
            for d in write_descs(base, rows_v):
                d.wait()

        load_x(tile_base, x0a, x1a, x2a)
        compute_idx(x0a, x1a, x2a, idx_a)

        @pl.loop(tile_base, tile_end, step=np.int32(2 * _C))
        def chunk_pair(base):
            not_first = base > tile_base

            @pl.when(not_first)
            def _():
                wait_writes(base - np.int32(2 * _C), rows_a)

            nb = base + np.int32(_C)
            load_x(nb, x0b, x1b, x2b)
            cpa = pltpu.async_copy(table_hbm.at[idx_a], rows_a, semg)
            compute_idx(x0b, x1b, x2b, idx_b)
            cpa.wait()
            issue_writes(base, rows_a)

            @pl.when(not_first)
            def _():
                wait_writes(base - np.int32(_C), rows_b)

            na = jnp.minimum(base + np.int32(2 * _C), last_load)
            load_x(na, x0a, x1a, x2a)
            cpb = pltpu.async_copy(table_hbm.at[idx_b], rows_b, semg)
            compute_idx(x0a, x1a, x2a, idx_a)
            cpb.wait()
            issue_writes(nb, rows_b)

        wait_writes(tile_end - np.int32(2 * _C), rows_a)
        wait_writes(tile_end - np.int32(_C), rows_b)

    return k(x0, x1, x2, table)


def kernel(x, embeddings):
    x = x.astype(jnp.float32)
    tw = (embeddings.astype(jnp.float32)
          .reshape(_NUM_LEVELS, _HASH_SIZE // 128, 128, 2)
          .transpose(0, 1, 3, 2)
          .reshape(_TW))
    out1d = _sc_lookup(x[:, 0], x[:, 1], x[:, 2], tw)
    return (out1d.reshape(_TRS, _N // 128, 8, 128)
            .transpose(1, 3, 0, 2)
            .reshape(_N, _F))
